# Initial kernel scaffold; baseline (speedup 1.0000x reference)
import jax
import jax.numpy as jnp
from jax.experimental import pallas as pl


def kernel(x, pe):
    raise NotImplementedError("write your pallas kernel here")


# speedup vs baseline: 2.1566x; 2.1566x over previous
import functools
import math

import jax
import jax.numpy as jnp
from jax.experimental import pallas as pl


def _add_pe_block(x_ref, pe_ref, o_ref, *, inv_scale):
    o_ref[...] = x_ref[...] + pe_ref[...] * inv_scale


def kernel(x, pe):
    B, L, D = x.shape
    inv_scale = 1.0 / math.sqrt(D)

    bs = 256
    while L % bs != 0:
        bs //= 2

    return pl.pallas_call(
        functools.partial(_add_pe_block, inv_scale=inv_scale),
        grid=(L // bs,),
        in_specs=[
            pl.BlockSpec((B, bs, D), lambda s: (0, s, 0)),
            pl.BlockSpec((bs, D), lambda s: (s, 0)),
        ],
        out_specs=pl.BlockSpec((B, bs, D), lambda s: (0, s, 0)),
        out_shape=jax.ShapeDtypeStruct((B, L, D), x.dtype),
    )(x, pe[:L])


# speedup vs baseline: 2.1672x; 1.0049x over previous
import functools
import math

import jax
import jax.numpy as jnp
from jax.experimental import pallas as pl


def _add_pe_block(x_ref, pe_ref, o_ref, *, inv_scale):
    o_ref[...] = x_ref[...] + pe_ref[...] * inv_scale


def kernel(x, pe):
    B, L, D = x.shape
    inv_scale = 1.0 / math.sqrt(D)

    bs = 512
    while L % bs != 0:
        bs //= 2

    return pl.pallas_call(
        functools.partial(_add_pe_block, inv_scale=inv_scale),
        grid=(L // bs,),
        in_specs=[
            pl.BlockSpec((B, bs, D), lambda s: (0, s, 0)),
            pl.BlockSpec((bs, D), lambda s: (s, 0)),
        ],
        out_specs=pl.BlockSpec((B, bs, D), lambda s: (0, s, 0)),
        out_shape=jax.ShapeDtypeStruct((B, L, D), x.dtype),
    )(x, pe[:L])
